# baseline (speedup 1.0000x reference)
import functools

import jax
import jax.numpy as jnp
from jax import lax
from jax.experimental import pallas as pl
from jax.experimental.pallas import tpu as pltpu
from jax.experimental.pallas import tpu_sc as plsc

_B, _L, _NB, _LB, _D, _V = 1024, 200, 5, 50, 128, 100000
_NEG = _NB * _LB
_NEGP = 256
_LP = 208
_NC, _NS, _LANES = 2, 16, 16
_NW = _NC * _NS
_RPW = _B // _NW
_C = _D // _LANES
_NH0, _NH1 = 128, _NEG - 128


def _zeros8():
  return tuple(jnp.zeros((_LANES,), jnp.float32) for _ in range(_C))


def _sc_uae(tokens, negs, table, wt, bias):
  mesh = plsc.VectorSubcoreMesh(core_axis_name="c", subcore_axis_name="s")

  @functools.partial(
      pl.kernel,
      out_type=(
          jax.ShapeDtypeStruct((_B, _D), jnp.float32),
          jax.ShapeDtypeStruct((_B, _D), jnp.float32),
      ),
      mesh=mesh,
      compiler_params=pltpu.CompilerParams(needs_layout_passes=False),
      scratch_types=[
          pltpu.VMEM((_RPW, _L), jnp.int32),
          pltpu.VMEM((_RPW, _NEGP), jnp.int32),
          pltpu.VMEM((_LP, _D), jnp.float32),
          pltpu.VMEM((_LP, _D), jnp.float32),
          pltpu.VMEM((_NH0, _D), jnp.float32),
          pltpu.VMEM((_NH1, _D), jnp.float32),
          pltpu.VMEM((_D, _D), jnp.float32),
          pltpu.VMEM((_D,), jnp.float32),
          pltpu.VMEM((_D,), jnp.float32),
          pltpu.VMEM((_LP,), jnp.float32),
          pltpu.VMEM((_LP,), jnp.float32),
          pltpu.VMEM((_D,), jnp.float32),
          pltpu.VMEM((_D,), jnp.float32),
          pltpu.SemaphoreType.DMA,
          pltpu.SemaphoreType.DMA,
      ],
  )
  def k(tok_hbm, neg_hbm, tab_hbm, wt_hbm, b_hbm, out_hbm, nout_hbm,
        tok_idx, neg_idx, ebuf0, ebuf1, nbuf0, nbuf1, wt_v, b_v, m_v, e_v,
        w_v, o_v, no_v, sem_t, sem_n):
    wid = lax.axis_index("s") * _NC + lax.axis_index("c")
    base = wid * _RPW
    ebufs = (ebuf0, ebuf1)

    pltpu.sync_copy(wt_hbm, wt_v)
    pltpu.sync_copy(b_hbm, b_v)
    pltpu.sync_copy(tok_hbm.at[pl.ds(base, _RPW)], tok_idx)
    pltpu.sync_copy(neg_hbm.at[pl.ds(base, _RPW)], neg_idx)

    def tok_gather(i, buf):
      pltpu.async_copy(tab_hbm.at[tok_idx.at[i, pl.ds(0, 128)]],
                       buf.at[pl.ds(0, 128)], sem_t)
      pltpu.async_copy(tab_hbm.at[tok_idx.at[i, pl.ds(128, _L - 128)]],
                       buf.at[pl.ds(128, _L - 128)], sem_t)

    def tok_wait(buf):
      pltpu.make_async_copy(tab_hbm.at[tok_idx.at[0, pl.ds(0, 128)]],
                            buf.at[pl.ds(0, 128)], sem_t).wait()
      pltpu.make_async_copy(tab_hbm.at[tok_idx.at[0, pl.ds(128, _L - 128)]],
                            buf.at[pl.ds(128, _L - 128)], sem_t).wait()

    lane = lax.iota(jnp.int32, _LANES)

    tok_gather(0, ebuf0)

    def row_body(i, erows, enext):
      cn0 = pltpu.async_copy(tab_hbm.at[neg_idx.at[i, pl.ds(0, _NH0)]],
                             nbuf0, sem_n)
      cn1 = pltpu.async_copy(tab_hbm.at[neg_idx.at[i, pl.ds(_NH0, _NH1)]],
                             nbuf1, sem_n)
      tok_wait(erows)
      inext = jnp.minimum(i + 1, _RPW - 1)
      tok_gather(inext, enext)

      def mean_body(lb, acc):
        for u in range(8):
          l = lb * 8 + u
          acc = tuple(acc[c] + erows[l, pl.ds(c * _LANES, _LANES)]
                      for c in range(_C))
        return acc
      acc = lax.fori_loop(0, _L // 8, mean_body, _zeros8())
      for c in range(_C):
        m_v[pl.ds(c * _LANES, _LANES)] = acc[c] * (1.0 / _L)

      def mv_body(jc, y):
        m16 = m_v[pl.ds(jc * _LANES, _LANES)]
        for kk in range(_LANES):
          j = jc * _LANES + kk
          y = tuple(y[c] + m16[kk] * wt_v[j, pl.ds(c * _LANES, _LANES)]
                    for c in range(_C))
        return y
      y = lax.fori_loop(0, _D // _LANES, mv_body, _zeros8())
      y = tuple(y[c] + b_v[pl.ds(c * _LANES, _LANES)] for c in range(_C))

      def eij_chunk(lc, carry):
        row_ids = lane + lc * _LANES
        e16 = jnp.zeros((_LANES,), jnp.float32)
        for c in range(_C):
          a = jnp.zeros((_LANES,), jnp.float32)
          for kk in range(_LANES):
            d = c * _LANES + kk
            col = jnp.full((_LANES,), d, jnp.int32)
            a = a + plsc.load_gather(erows, [row_ids, col]) * y[c][kk]
          e16 = e16 + a
        e_v[pl.ds(lc * _LANES, _LANES)] = e16
        return carry
      lax.fori_loop(0, _LP // _LANES, eij_chunk, 0)

      sacc = jnp.zeros((_LANES,), jnp.float32)
      for c in range(_LP // _LANES):
        x = e_v[pl.ds(c * _LANES, _LANES)]
        e2x = jnp.exp(x * 2.0)
        t = 1.0 - 2.0 / (e2x + 1.0)
        p = jnp.exp(t)
        if (c + 1) * _LANES > _L:
          p = jnp.where(lane < _L - c * _LANES, p, jnp.zeros_like(p))
        w_v[pl.ds(c * _LANES, _LANES)] = p
        sacc = sacc + p
      s = sacc[0]
      for kk in range(1, _LANES):
        s = s + sacc[kk]
      rs = 1.0 / jnp.broadcast_to(s, (_LANES,))

      def ws_chunk(lc, acc):
        w16 = w_v[pl.ds(lc * _LANES, _LANES)]
        for kk in range(_LANES):
          l = lc * _LANES + kk
          acc = tuple(acc[c] + w16[kk] * erows[l, pl.ds(c * _LANES, _LANES)]
                      for c in range(_C))
        return acc
      oacc = lax.fori_loop(0, _L // _LANES, ws_chunk, _zeros8())
      w16 = w_v[pl.ds((_L // _LANES) * _LANES, _LANES)]
      for kk in range(_L - (_L // _LANES) * _LANES):
        l = (_L // _LANES) * _LANES + kk
        oacc = tuple(oacc[c] + w16[kk] * erows[l, pl.ds(c * _LANES, _LANES)]
                     for c in range(_C))
      for c in range(_C):
        o_v[pl.ds(c * _LANES, _LANES)] = oacc[c] * rs
      pltpu.sync_copy(o_v, out_hbm.at[base + i])

      cn0.wait()
      cn1.wait()
      def neg0_body(nb, acc):
        for u in range(8):
          n = nb * 8 + u
          acc = tuple(acc[c] + nbuf0[n, pl.ds(c * _LANES, _LANES)]
                      for c in range(_C))
        return acc
      nacc = lax.fori_loop(0, _NH0 // 8, neg0_body, _zeros8())
      def neg1_body(nb, acc):
        for u in range(8):
          n = nb * 8 + u
          acc = tuple(acc[c] + nbuf1[n, pl.ds(c * _LANES, _LANES)]
                      for c in range(_C))
        return acc
      nacc = lax.fori_loop(0, _NH1 // 8, neg1_body, nacc)
      for n in range((_NH1 // 8) * 8, _NH1):
        nacc = tuple(nacc[c] + nbuf1[n, pl.ds(c * _LANES, _LANES)]
                     for c in range(_C))
      for c in range(_C):
        no_v[pl.ds(c * _LANES, _LANES)] = nacc[c] * (1.0 / _NEG)
      pltpu.sync_copy(no_v, nout_hbm.at[base + i])

    def pair_body(p, carry):
      row_body(2 * p, ebuf0, ebuf1)
      row_body(2 * p + 1, ebuf1, ebuf0)
      return carry
    lax.fori_loop(0, _RPW // 2, pair_body, 0)

    tok_wait(ebuf0)

  return k(tokens, negs, table, wt, bias)


def kernel(tokens, sentence_embs, neg_bags, token_embedding, att_W, att_b):
  negs = jnp.pad(neg_bags.reshape(_B, _NEG).astype(jnp.int32),
                 ((0, 0), (0, _NEGP - _NEG)))
  out, nmean = _sc_uae(tokens.astype(jnp.int32), negs, token_embedding,
                       att_W.T, att_b)
  return out, nmean, sentence_embs


# speedup vs baseline: 2.6246x; 2.6246x over previous
import functools

import jax
import jax.numpy as jnp
from jax import lax
from jax.experimental import pallas as pl
from jax.experimental.pallas import tpu as pltpu
from jax.experimental.pallas import tpu_sc as plsc

_B, _L, _NB, _LB, _D, _V = 1024, 200, 5, 50, 128, 100000
_NEG = _NB * _LB
_NEGP = 256
_LP = 208
_NC, _NS, _LANES = 2, 16, 16
_NW = _NC * _NS
_RPW = _B // _NW
_C = _D // _LANES
_NH0, _NH1 = 128, _NEG - 128


def _zeros8():
  return tuple(jnp.zeros((_LANES,), jnp.float32) for _ in range(_C))


def _sc_uae(tokens, negs, table, wt, bias):
  mesh = plsc.VectorSubcoreMesh(core_axis_name="c", subcore_axis_name="s")

  @functools.partial(
      pl.kernel,
      out_type=(
          jax.ShapeDtypeStruct((_B, _D), jnp.float32),
          jax.ShapeDtypeStruct((_B, _D), jnp.float32),
      ),
      mesh=mesh,
      compiler_params=pltpu.CompilerParams(needs_layout_passes=False),
      scratch_types=[
          pltpu.VMEM((_RPW, _L), jnp.int32),
          pltpu.VMEM((_RPW, _NEGP), jnp.int32),
          pltpu.VMEM((_LP, _D), jnp.float32),
          pltpu.VMEM((_LP, _D), jnp.float32),
          pltpu.VMEM((_NH0, _D), jnp.float32),
          pltpu.VMEM((_NH1, _D), jnp.float32),
          pltpu.VMEM((_D, _D), jnp.float32),
          pltpu.VMEM((_D,), jnp.float32),
          pltpu.VMEM((_D,), jnp.float32),
          pltpu.VMEM((_LP,), jnp.float32),
          pltpu.VMEM((_LP,), jnp.float32),
          pltpu.VMEM((_D,), jnp.float32),
          pltpu.VMEM((_D,), jnp.float32),
          pltpu.SemaphoreType.DMA,
          pltpu.SemaphoreType.DMA,
      ],
  )
  def k(tok_hbm, neg_hbm, tab_hbm, wt_hbm, b_hbm, out_hbm, nout_hbm,
        tok_idx, neg_idx, ebuf0, ebuf1, nbuf0, nbuf1, wt_v, b_v, m_v, e_v,
        w_v, o_v, no_v, sem_t, sem_n):
    wid = lax.axis_index("s") * _NC + lax.axis_index("c")
    base = wid * _RPW
    ebufs = (ebuf0, ebuf1)

    pltpu.sync_copy(wt_hbm, wt_v)
    pltpu.sync_copy(b_hbm, b_v)
    pltpu.sync_copy(tok_hbm.at[pl.ds(base, _RPW)], tok_idx)
    pltpu.sync_copy(neg_hbm.at[pl.ds(base, _RPW)], neg_idx)

    def tok_gather(i, buf):
      pltpu.async_copy(tab_hbm.at[tok_idx.at[i, pl.ds(0, 128)]],
                       buf.at[pl.ds(0, 128)], sem_t)
      pltpu.async_copy(tab_hbm.at[tok_idx.at[i, pl.ds(128, _L - 128)]],
                       buf.at[pl.ds(128, _L - 128)], sem_t)

    def tok_wait(buf):
      pltpu.make_async_copy(tab_hbm.at[tok_idx.at[0, pl.ds(0, 128)]],
                            buf.at[pl.ds(0, 128)], sem_t).wait()
      pltpu.make_async_copy(tab_hbm.at[tok_idx.at[0, pl.ds(128, _L - 128)]],
                            buf.at[pl.ds(128, _L - 128)], sem_t).wait()

    lane = lax.iota(jnp.int32, _LANES)

    tok_gather(0, ebuf0)

    def row_body(i, erows, enext):
      cn0 = pltpu.async_copy(tab_hbm.at[neg_idx.at[i, pl.ds(0, _NH0)]],
                             nbuf0, sem_n)
      cn1 = pltpu.async_copy(tab_hbm.at[neg_idx.at[i, pl.ds(_NH0, _NH1)]],
                             nbuf1, sem_n)
      tok_wait(erows)
      inext = jnp.minimum(i + 1, _RPW - 1)
      tok_gather(inext, enext)

      def mean_body(lb, acc):
        for u in range(8):
          l = lb * 8 + u
          acc = tuple(acc[c] + erows[l, pl.ds(c * _LANES, _LANES)]
                      for c in range(_C))
        return acc
      acc = lax.fori_loop(0, _L // 8, mean_body, _zeros8())
      for c in range(_C):
        m_v[pl.ds(c * _LANES, _LANES)] = acc[c] * (1.0 / _L)

      def mv_body(jc, y):
        m16 = m_v[pl.ds(jc * _LANES, _LANES)]
        for kk in range(_LANES):
          j = jc * _LANES + kk
          y = tuple(y[c] + m16[kk] * wt_v[j, pl.ds(c * _LANES, _LANES)]
                    for c in range(_C))
        return y
      y = lax.fori_loop(0, _D // _LANES, mv_body, _zeros8())
      y = tuple(y[c] + b_v[pl.ds(c * _LANES, _LANES)] for c in range(_C))

      def eij_chunk(lc, carry):
        e16 = jnp.zeros((_LANES,), jnp.float32)
        for kk in range(_LANES):
          l = lc * _LANES + kk
          p = erows[l, pl.ds(0, _LANES)] * y[0]
          for c in range(1, _C):
            p = p + erows[l, pl.ds(c * _LANES, _LANES)] * y[c]
          e16 = jnp.where(lane == kk, jnp.sum(p), e16)
        e_v[pl.ds(lc * _LANES, _LANES)] = e16
        return carry
      lax.fori_loop(0, _L // _LANES, eij_chunk, 0)
      e16 = jnp.zeros((_LANES,), jnp.float32)
      for kk in range(_L - (_L // _LANES) * _LANES):
        l = (_L // _LANES) * _LANES + kk
        p = erows[l, pl.ds(0, _LANES)] * y[0]
        for c in range(1, _C):
          p = p + erows[l, pl.ds(c * _LANES, _LANES)] * y[c]
        e16 = jnp.where(lane == kk, jnp.sum(p), e16)
      e_v[pl.ds((_L // _LANES) * _LANES, _LANES)] = e16

      sacc = jnp.zeros((_LANES,), jnp.float32)
      for c in range(_LP // _LANES):
        x = e_v[pl.ds(c * _LANES, _LANES)]
        e2x = jnp.exp(x * 2.0)
        t = 1.0 - 2.0 / (e2x + 1.0)
        p = jnp.exp(t)
        if (c + 1) * _LANES > _L:
          p = jnp.where(lane < _L - c * _LANES, p, jnp.zeros_like(p))
        w_v[pl.ds(c * _LANES, _LANES)] = p
        sacc = sacc + p
      s = sacc[0]
      for kk in range(1, _LANES):
        s = s + sacc[kk]
      rs = 1.0 / jnp.broadcast_to(s, (_LANES,))

      def ws_chunk(lc, acc):
        w16 = w_v[pl.ds(lc * _LANES, _LANES)]
        for kk in range(_LANES):
          l = lc * _LANES + kk
          acc = tuple(acc[c] + w16[kk] * erows[l, pl.ds(c * _LANES, _LANES)]
                      for c in range(_C))
        return acc
      oacc = lax.fori_loop(0, _L // _LANES, ws_chunk, _zeros8())
      w16 = w_v[pl.ds((_L // _LANES) * _LANES, _LANES)]
      for kk in range(_L - (_L // _LANES) * _LANES):
        l = (_L // _LANES) * _LANES + kk
        oacc = tuple(oacc[c] + w16[kk] * erows[l, pl.ds(c * _LANES, _LANES)]
                     for c in range(_C))
      for c in range(_C):
        o_v[pl.ds(c * _LANES, _LANES)] = oacc[c] * rs
      pltpu.sync_copy(o_v, out_hbm.at[base + i])

      cn0.wait()
      cn1.wait()
      def neg0_body(nb, acc):
        for u in range(8):
          n = nb * 8 + u
          acc = tuple(acc[c] + nbuf0[n, pl.ds(c * _LANES, _LANES)]
                      for c in range(_C))
        return acc
      nacc = lax.fori_loop(0, _NH0 // 8, neg0_body, _zeros8())
      def neg1_body(nb, acc):
        for u in range(8):
          n = nb * 8 + u
          acc = tuple(acc[c] + nbuf1[n, pl.ds(c * _LANES, _LANES)]
                      for c in range(_C))
        return acc
      nacc = lax.fori_loop(0, _NH1 // 8, neg1_body, nacc)
      for n in range((_NH1 // 8) * 8, _NH1):
        nacc = tuple(nacc[c] + nbuf1[n, pl.ds(c * _LANES, _LANES)]
                     for c in range(_C))
      for c in range(_C):
        no_v[pl.ds(c * _LANES, _LANES)] = nacc[c] * (1.0 / _NEG)
      pltpu.sync_copy(no_v, nout_hbm.at[base + i])

    def pair_body(p, carry):
      row_body(2 * p, ebuf0, ebuf1)
      row_body(2 * p + 1, ebuf1, ebuf0)
      return carry
    lax.fori_loop(0, _RPW // 2, pair_body, 0)

    tok_wait(ebuf0)

  return k(tokens, negs, table, wt, bias)


def kernel(tokens, sentence_embs, neg_bags, token_embedding, att_W, att_b):
  negs = jnp.pad(neg_bags.reshape(_B, _NEG).astype(jnp.int32),
                 ((0, 0), (0, _NEGP - _NEG)))
  out, nmean = _sc_uae(tokens.astype(jnp.int32), negs, token_embedding,
                       att_W.T, att_b)
  return out, nmean, sentence_embs
